# baseline (speedup 1.0000x reference)
import functools
import jax
import jax.numpy as jnp
from jax import lax
from jax.experimental import pallas as pl
from jax.experimental.pallas import tpu as pltpu
from jax.experimental.pallas import tpu_sc as plsc

B, W, D = 32, 168, 128
HID = 256
M = 5
OUT = 128
EMB = 64
E = 1024
K = 2
G4 = 4 * HID
TCH = 24
NCH = W // TCH
RCH = B * TCH


_SC_MESH = plsc.VectorSubcoreMesh(core_axis_name="c", subcore_axis_name="s")


@functools.partial(
    pl.kernel,
    mesh=_SC_MESH,
    compiler_params=pltpu.CompilerParams(needs_layout_passes=False),
    out_type=jax.ShapeDtypeStruct((B, W, W), jnp.float32),
    scratch_types=[
        pltpu.VMEM((E,), jnp.int32),
        pltpu.VMEM((E,), jnp.int32),
        pltpu.VMEM((E,), jnp.float32),
        pltpu.VMEM((W, W), jnp.float32),
    ],
)
def _sc_adj_build(tei_hbm, tew_hbm, zer_hbm, out_hbm, src_v, dst_v, ew_v,
                  acc_v):
    b = lax.axis_index("s") * 2 + lax.axis_index("c")
    pltpu.sync_copy(tei_hbm.at[b, 0], src_v)
    pltpu.sync_copy(tei_hbm.at[b, 1], dst_v)
    pltpu.sync_copy(tew_hbm.at[b], ew_v)
    pltpu.sync_copy(zer_hbm, acc_v)

    def scat_body(i, carry):
        s = src_v[pl.ds(i * 16, 16)]
        d = dst_v[pl.ds(i * 16, 16)]
        w = ew_v[pl.ds(i * 16, 16)]
        plsc.addupdate_scatter(acc_v, [s, d], w)
        return carry

    lax.fori_loop(0, E // 16, scat_body, 0, unroll=4)
    pltpu.sync_copy(acc_v, out_hbm.at[b])


def _fused_body(x_ref, a_ref, n1_ref, n2_ref, wt_ref, ws_ref, wih_ref,
                bt_ref, bs_ref, bg_ref, whh_ref, wmu_ref, bmu_ref, wsig_ref,
                bsig_ref, wpi_ref, bpi_ref,
                mu_ref, sg_ref, pi_ref,
                mf_scr, mb_scr, g_scr, h_scr, c_scr, hs_scr):
    i = pl.program_id(0)

    def mmT(a, b):
        return jax.lax.dot_general(a, b, (((0,), (0,)), ((), ())),
                                   preferred_element_type=jnp.float32)

    def mm(a, b):
        return jax.lax.dot_general(a, b, (((1,), (0,)), ((), ())),
                                   preferred_element_type=jnp.float32)

    def mmBT(a, b):
        return jax.lax.dot_general(a, b, (((1,), (1,)), ((), ())),
                                   preferred_element_type=jnp.float32)

    @pl.when(i == 0)
    def _():
        z = jnp.maximum(jnp.dot(n1_ref[...], n2_ref[...],
                                preferred_element_type=jnp.float32), 0.0)
        zm = jnp.max(z, axis=1, keepdims=True)
        ez = jnp.exp(z - zm)
        adp = ez / jnp.sum(ez, axis=1, keepdims=True)
        colsum = jnp.sum(adp, axis=0, keepdims=True)
        rowsum = jnp.sum(adp, axis=1, keepdims=True)
        mf_scr[...] = adp / jnp.maximum(colsum, 1e-8)
        mb_scr[...] = (adp / jnp.maximum(rowsum, 1e-8)).T

    @pl.when(i < B)
    def _phase1():
        xb = x_ref[0]
        a_raw = a_ref[0]

        ones = jnp.ones((W, 1), jnp.float32)
        cf = jax.lax.dot_general(a_raw, ones, (((0,), (0,)), ((), ())))
        cb = jax.lax.dot_general(a_raw, ones, (((1,), (0,)), ((), ())))
        rf = 1.0 / jnp.maximum(cf, 1e-8)
        rb = 1.0 / jnp.maximum(cb, 1e-8)

        t1 = mmT(a_raw, xb) * rf
        t2 = mmT(a_raw, t1) * rf
        t3 = mm(a_raw, xb) * rb
        t4 = mm(a_raw, t3) * rb
        dt = (mm(t1, wt_ref[0]) + mm(t2, wt_ref[1]) + mm(t3, wt_ref[2])
              + mm(t4, wt_ref[3]) + bt_ref[...])

        mf = mf_scr[...]
        mb = mb_scr[...]
        u1 = mm(xb, mf)
        u2 = mm(u1, mf)
        u3 = mm(xb, mb)
        u4 = mm(u3, mb)
        ds = (mmT(ws_ref[0], u1) + mmT(ws_ref[1], u2) + mmT(ws_ref[2], u3)
              + mmT(ws_ref[3], u4) + bs_ref[...])

        wih = wih_ref[...]
        g = (mmBT(dt, wih[:, 0:HID]) + mmBT(ds, wih[:, HID:HID + D])
             + mmBT(xb, wih[:, HID + D:]) + bg_ref[...])
        for cc in range(NCH):
            g_scr[cc, i] = g[cc * TCH:(cc + 1) * TCH, :]

    @pl.when(i == B)
    def _():
        h_scr[...] = jnp.zeros((B, HID), jnp.float32)
        c_scr[...] = jnp.zeros((B, HID), jnp.float32)

    @pl.when(i >= B)
    def _phase2():
        cix = i - B
        h = h_scr[...]
        c = c_scr[...]
        whh = whh_ref[...]
        for t in range(TCH):
            gates = g_scr[cix, :, t, :] + jax.lax.dot_general(
                h.astype(jnp.bfloat16), whh, (((1,), (1,)), ((), ())),
                preferred_element_type=jnp.float32)
            ig = jax.nn.sigmoid(gates[:, 0:HID])
            fg = jax.nn.sigmoid(gates[:, HID:2 * HID])
            gg = jnp.tanh(gates[:, 2 * HID:3 * HID])
            og = jax.nn.sigmoid(gates[:, 3 * HID:])
            c = fg * c + ig * gg
            h = og * jnp.tanh(c)
            hs_scr[:, t, :] = h
        h_scr[...] = h
        c_scr[...] = c

        hv = hs_scr[...].reshape(RCH, HID)
        zmu = jnp.dot(hv, wmu_ref[...],
                      preferred_element_type=jnp.float32) + bmu_ref[...]
        zsg = jnp.exp(jnp.dot(hv, wsig_ref[...],
                              preferred_element_type=jnp.float32)
                      + bsig_ref[...])
        zmu3 = zmu.reshape(B, TCH, M * OUT)
        zsg3 = zsg.reshape(B, TCH, M * OUT)
        for m in range(M):
            mu_ref[:, :, m, :] = zmu3[:, :, m * OUT:(m + 1) * OUT]
            sg_ref[:, :, m, :] = zsg3[:, :, m * OUT:(m + 1) * OUT]
        z = jnp.dot(hv, wpi_ref[...],
                    preferred_element_type=jnp.float32) + bpi_ref[...]
        zm = jnp.max(z, axis=1, keepdims=True)
        ez = jnp.exp(z - zm)
        pi_ref[...] = (ez / jnp.sum(ez, axis=1, keepdims=True))[:, :M].reshape(
            B, TCH, M)


def _fused(x, a3, n1, n2, wt_r, ws_r, wih, bt_row, bs_col, bg_row, whh,
           wmu, bmu_row, wsig, bsig_row, wpi_pad, bpi_pad):
    const = lambda shp: pl.BlockSpec(shp, lambda i: (0,) * len(shp))
    samp = lambda i: (jnp.minimum(i, B - 1), 0, 0)
    outc4 = lambda i: (0, jnp.maximum(i - B, 0), 0, 0)
    outc3 = lambda i: (0, jnp.maximum(i - B, 0), 0)
    return pl.pallas_call(
        _fused_body,
        grid=(B + NCH,),
        in_specs=[
            pl.BlockSpec((1, W, D), samp),
            pl.BlockSpec((1, W, W), samp),
            const((D, EMB)), const((EMB, D)),
            const((4, D, HID)), const((4, W, W)),
            const((G4, 2 * D + HID)),
            const((1, HID)), const((W, 1)), const((1, G4)),
            const((G4, HID)),
            const((HID, M * OUT)), const((1, M * OUT)),
            const((HID, M * OUT)), const((1, M * OUT)),
            const((HID, 128)), const((1, 128)),
        ],
        out_specs=[
            pl.BlockSpec((B, TCH, M, OUT), outc4),
            pl.BlockSpec((B, TCH, M, OUT), outc4),
            pl.BlockSpec((B, TCH, M), outc3),
        ],
        out_shape=[
            jax.ShapeDtypeStruct((B, W, M, OUT), jnp.float32),
            jax.ShapeDtypeStruct((B, W, M, OUT), jnp.float32),
            jax.ShapeDtypeStruct((B, W, M), jnp.float32),
        ],
        scratch_shapes=[
            pltpu.VMEM((D, D), jnp.float32),
            pltpu.VMEM((D, D), jnp.float32),
            pltpu.VMEM((NCH, B, TCH, G4), jnp.float32),
            pltpu.VMEM((B, HID), jnp.float32),
            pltpu.VMEM((B, HID), jnp.float32),
            pltpu.VMEM((B, TCH, HID), jnp.float32),
        ],
    )(x, a3, n1, n2, wt_r, ws_r, wih, bt_row, bs_col, bg_row, whh,
      wmu, bmu_row, wsig, bsig_row, wpi_pad, bpi_pad)


def kernel(x, temporal_edge_i, temporal_edge_w, params):
    p = params
    tei = temporal_edge_i.astype(jnp.int32)

    a3 = _sc_adj_build(tei, temporal_edge_w, jnp.zeros((W, W), jnp.float32))

    wt_r = p['Wt'].reshape(2 * K, D, HID)
    ws_r = p['Ws'].reshape(2 * K, W, W)
    bt_row = p['bt'].reshape(1, HID)
    bs_col = p['bs'].reshape(W, 1)
    bg_row = (p['bih'] + p['bhh']).reshape(1, G4)
    wpi_pad = jnp.zeros((HID, 128), jnp.float32).at[:, :M].set(p['Wpi'])
    bpi_pad = jnp.full((1, 128), -1e30, jnp.float32).at[0, :M].set(p['bpi'])

    mu, sigma, pi = _fused(
        x, a3, p['N1'], p['N2'], wt_r, ws_r, p['Wih'], bt_row, bs_col,
        bg_row, p['Whh'].astype(jnp.bfloat16), p['Wmu'],
        p['bmu'].reshape(1, M * OUT),
        p['Wsig'], p['bsig'].reshape(1, M * OUT), wpi_pad, bpi_pad)
    return (mu, sigma, pi)


# speedup vs baseline: 1.0173x; 1.0173x over previous
import functools
import jax
import jax.numpy as jnp
from jax import lax
from jax.experimental import pallas as pl
from jax.experimental.pallas import tpu as pltpu
from jax.experimental.pallas import tpu_sc as plsc

B, W, D = 32, 168, 128
HID = 256
M = 5
OUT = 128
EMB = 64
E = 1024
K = 2
G4 = 4 * HID
TCH = 24
NCH = W // TCH
RCH = B * TCH


_SC_MESH = plsc.VectorSubcoreMesh(core_axis_name="c", subcore_axis_name="s")


@functools.partial(
    pl.kernel,
    mesh=_SC_MESH,
    compiler_params=pltpu.CompilerParams(needs_layout_passes=False),
    out_type=jax.ShapeDtypeStruct((B, W, W), jnp.float32),
    scratch_types=[
        pltpu.VMEM((E,), jnp.int32),
        pltpu.VMEM((E,), jnp.int32),
        pltpu.VMEM((E,), jnp.float32),
        pltpu.VMEM((W, W), jnp.float32),
    ],
)
def _sc_adj_build(tei_hbm, tew_hbm, zer_hbm, out_hbm, src_v, dst_v, ew_v,
                  acc_v):
    b = lax.axis_index("s") * 2 + lax.axis_index("c")
    pltpu.sync_copy(tei_hbm.at[b, 0], src_v)
    pltpu.sync_copy(tei_hbm.at[b, 1], dst_v)
    pltpu.sync_copy(tew_hbm.at[b], ew_v)
    pltpu.sync_copy(zer_hbm, acc_v)

    def scat_body(i, carry):
        s = src_v[pl.ds(i * 16, 16)]
        d = dst_v[pl.ds(i * 16, 16)]
        w = ew_v[pl.ds(i * 16, 16)]
        plsc.addupdate_scatter(acc_v, [s, d], w)
        return carry

    lax.fori_loop(0, E // 16, scat_body, 0, unroll=4)
    pltpu.sync_copy(acc_v, out_hbm.at[b])


def _fused_body(x_ref, a_ref, n1_ref, n2_ref, wt_ref, ws_ref, wih_ref,
                bt_ref, bs_ref, bg_ref, whh_ref, wmu_ref, bmu_ref, wsig_ref,
                bsig_ref, wpi_ref, bpi_ref,
                mu_ref, sg_ref, pi_ref,
                mf_scr, mb_scr, g_scr, h_scr, c_scr, hs_scr):
    i = pl.program_id(0)

    def mmT(a, b):
        return jax.lax.dot_general(a, b, (((0,), (0,)), ((), ())),
                                   preferred_element_type=jnp.float32)

    def mm(a, b):
        return jax.lax.dot_general(a, b, (((1,), (0,)), ((), ())),
                                   preferred_element_type=jnp.float32)

    def mmBT(a, b):
        return jax.lax.dot_general(a, b, (((1,), (1,)), ((), ())),
                                   preferred_element_type=jnp.float32)

    @pl.when(i == 0)
    def _():
        z = jnp.maximum(jnp.dot(n1_ref[...], n2_ref[...],
                                preferred_element_type=jnp.float32), 0.0)
        zm = jnp.max(z, axis=1, keepdims=True)
        ez = jnp.exp(z - zm)
        adp = ez / jnp.sum(ez, axis=1, keepdims=True)
        colsum = jnp.sum(adp, axis=0, keepdims=True)
        rowsum = jnp.sum(adp, axis=1, keepdims=True)
        mf_scr[...] = adp / jnp.maximum(colsum, 1e-8)
        mb_scr[...] = (adp / jnp.maximum(rowsum, 1e-8)).T

    @pl.when(i < B)
    def _phase1():
        xb = x_ref[0]
        a_raw = a_ref[0]

        ones = jnp.ones((W, 1), jnp.float32)
        cf = jax.lax.dot_general(a_raw, ones, (((0,), (0,)), ((), ())))
        cb = jax.lax.dot_general(a_raw, ones, (((1,), (0,)), ((), ())))
        rf = 1.0 / jnp.maximum(cf, 1e-8)
        rb = 1.0 / jnp.maximum(cb, 1e-8)

        t1 = mmT(a_raw, xb) * rf
        t2 = mmT(a_raw, t1) * rf
        t3 = mm(a_raw, xb) * rb
        t4 = mm(a_raw, t3) * rb
        dt = (mm(t1, wt_ref[0]) + mm(t2, wt_ref[1]) + mm(t3, wt_ref[2])
              + mm(t4, wt_ref[3]) + bt_ref[...])

        mf = mf_scr[...]
        mb = mb_scr[...]
        u1 = mm(xb, mf)
        u2 = mm(u1, mf)
        u3 = mm(xb, mb)
        u4 = mm(u3, mb)
        ds = (mmT(ws_ref[0], u1) + mmT(ws_ref[1], u2) + mmT(ws_ref[2], u3)
              + mmT(ws_ref[3], u4) + bs_ref[...])

        wih = wih_ref[...]
        g = (mmBT(dt, wih[:, 0:HID]) + mmBT(ds, wih[:, HID:HID + D])
             + mmBT(xb, wih[:, HID + D:]) + bg_ref[...])
        for cc in range(NCH):
            g_scr[cc, i] = g[cc * TCH:(cc + 1) * TCH, :]

    @pl.when(i == B)
    def _():
        h_scr[...] = jnp.zeros((B, HID), jnp.float32)
        c_scr[...] = jnp.zeros((B, HID), jnp.float32)

    @pl.when(i >= B)
    def _phase2():
        cix = i - B
        h = h_scr[...]
        c = c_scr[...]
        whh = whh_ref[...]
        for t in range(TCH):
            gates = g_scr[cix, :, t, :] + jax.lax.dot_general(
                h, whh, (((1,), (1,)), ((), ())),
                preferred_element_type=jnp.float32)
            ig = jax.nn.sigmoid(gates[:, 0:HID])
            fg = jax.nn.sigmoid(gates[:, HID:2 * HID])
            gg = jnp.tanh(gates[:, 2 * HID:3 * HID])
            og = jax.nn.sigmoid(gates[:, 3 * HID:])
            c = fg * c + ig * gg
            h = og * jnp.tanh(c)
            hs_scr[:, t, :] = h
        h_scr[...] = h
        c_scr[...] = c

        hv = hs_scr[...].reshape(RCH, HID)
        zmu = jnp.dot(hv, wmu_ref[...],
                      preferred_element_type=jnp.float32) + bmu_ref[...]
        zsg = jnp.exp(jnp.dot(hv, wsig_ref[...],
                              preferred_element_type=jnp.float32)
                      + bsig_ref[...])
        zmu3 = zmu.reshape(B, TCH, M * OUT)
        zsg3 = zsg.reshape(B, TCH, M * OUT)
        for m in range(M):
            mu_ref[:, :, m, :] = zmu3[:, :, m * OUT:(m + 1) * OUT]
            sg_ref[:, :, m, :] = zsg3[:, :, m * OUT:(m + 1) * OUT]
        z = jnp.dot(hv, wpi_ref[...],
                    preferred_element_type=jnp.float32) + bpi_ref[...]
        zm = jnp.max(z, axis=1, keepdims=True)
        ez = jnp.exp(z - zm)
        pi_ref[...] = (ez / jnp.sum(ez, axis=1, keepdims=True))[:, :M].reshape(
            B, TCH, M)


def _fused(x, a3, n1, n2, wt_r, ws_r, wih, bt_row, bs_col, bg_row, whh,
           wmu, bmu_row, wsig, bsig_row, wpi_pad, bpi_pad):
    const = lambda shp: pl.BlockSpec(shp, lambda i: (0,) * len(shp))
    samp = lambda i: (jnp.minimum(i, B - 1), 0, 0)
    outc4 = lambda i: (0, jnp.maximum(i - B, 0), 0, 0)
    outc3 = lambda i: (0, jnp.maximum(i - B, 0), 0)
    return pl.pallas_call(
        _fused_body,
        grid=(B + NCH,),
        in_specs=[
            pl.BlockSpec((1, W, D), samp),
            pl.BlockSpec((1, W, W), samp),
            const((D, EMB)), const((EMB, D)),
            const((4, D, HID)), const((4, W, W)),
            const((G4, 2 * D + HID)),
            const((1, HID)), const((W, 1)), const((1, G4)),
            const((G4, HID)),
            const((HID, M * OUT)), const((1, M * OUT)),
            const((HID, M * OUT)), const((1, M * OUT)),
            const((HID, 128)), const((1, 128)),
        ],
        out_specs=[
            pl.BlockSpec((B, TCH, M, OUT), outc4),
            pl.BlockSpec((B, TCH, M, OUT), outc4),
            pl.BlockSpec((B, TCH, M), outc3),
        ],
        out_shape=[
            jax.ShapeDtypeStruct((B, W, M, OUT), jnp.float32),
            jax.ShapeDtypeStruct((B, W, M, OUT), jnp.float32),
            jax.ShapeDtypeStruct((B, W, M), jnp.float32),
        ],
        scratch_shapes=[
            pltpu.VMEM((D, D), jnp.float32),
            pltpu.VMEM((D, D), jnp.float32),
            pltpu.VMEM((NCH, B, TCH, G4), jnp.float32),
            pltpu.VMEM((B, HID), jnp.float32),
            pltpu.VMEM((B, HID), jnp.float32),
            pltpu.VMEM((B, TCH, HID), jnp.float32),
        ],
    )(x, a3, n1, n2, wt_r, ws_r, wih, bt_row, bs_col, bg_row, whh,
      wmu, bmu_row, wsig, bsig_row, wpi_pad, bpi_pad)


def kernel(x, temporal_edge_i, temporal_edge_w, params):
    p = params
    tei = temporal_edge_i.astype(jnp.int32)

    a3 = _sc_adj_build(tei, temporal_edge_w, jnp.zeros((W, W), jnp.float32))

    wt_r = p['Wt'].reshape(2 * K, D, HID)
    ws_r = p['Ws'].reshape(2 * K, W, W)
    bt_row = p['bt'].reshape(1, HID)
    bs_col = p['bs'].reshape(W, 1)
    bg_row = (p['bih'] + p['bhh']).reshape(1, G4)
    wpi_pad = jnp.zeros((HID, 128), jnp.float32).at[:, :M].set(p['Wpi'])
    bpi_pad = jnp.full((1, 128), -1e30, jnp.float32).at[0, :M].set(p['bpi'])

    mu, sigma, pi = _fused(
        x, a3, p['N1'], p['N2'], wt_r, ws_r, p['Wih'], bt_row, bs_col,
        bg_row, p['Whh'], p['Wmu'], p['bmu'].reshape(1, M * OUT),
        p['Wsig'], p['bsig'].reshape(1, M * OUT), wpi_pad, bpi_pad)
    return (mu, sigma, pi)
